# baseline (speedup 1.0000x reference)
import functools

import jax
import jax.numpy as jnp
from jax.experimental import pallas as pl
from jax.experimental.pallas import tpu as pltpu


def _body(a_ref, q_ref, i_ref, s_ref, o_ref, *, rep2, nb):
    q = q_ref[...]
    srep = jnp.repeat(s_ref[...], rep2, axis=0)
    b = ((q - 8).astype(jnp.float32) * srep).astype(jnp.bfloat16)
    ic = i_ref[...].astype(jnp.bfloat16)
    zero = jnp.zeros((), jnp.bfloat16)
    planes = [jnp.where(ic == p, b, zero) for p in range(4)]
    w = jnp.concatenate(planes, axis=0)
    o_ref[...] = jnp.dot(a_ref[...], w, preferred_element_type=jnp.float32)


def kernel(A, qvals, idx, s):
    M, K = A.shape
    K2, N = qvals.shape
    g4 = K2 // 2
    sg = s.shape[0]
    rep2 = K2 // sg
    nb = 256 if N % 256 == 0 else N

    Ac = jnp.transpose(A.reshape(M, g4, 4), (0, 2, 1)).reshape(M, K)
    A2 = jnp.repeat(Ac, 2, axis=1).astype(jnp.bfloat16)
    ic = jnp.transpose(idx, (0, 2, 1)).reshape(K2, N).astype(jnp.int16)

    body = functools.partial(_body, rep2=rep2, nb=nb)
    return pl.pallas_call(
        body,
        out_shape=jax.ShapeDtypeStruct((M, N), jnp.float32),
        grid=(N // nb,),
        in_specs=[
            pl.BlockSpec((M, 4 * K2), lambda n: (0, 0)),
            pl.BlockSpec((K2, nb), lambda n: (0, n)),
            pl.BlockSpec((K2, nb), lambda n: (0, n)),
            pl.BlockSpec((sg, nb), lambda n: (0, n)),
        ],
        out_specs=pl.BlockSpec((M, nb), lambda n: (0, n)),
        compiler_params=pltpu.CompilerParams(
            dimension_semantics=("parallel",),
            vmem_limit_bytes=50 * 1024 * 1024,
        ),
        name="sparse24_int4_matmul",
    )(A2, qvals, ic, s)


# speedup vs baseline: 1.0976x; 1.0976x over previous
import functools

import jax
import jax.numpy as jnp
from jax.experimental import pallas as pl
from jax.experimental.pallas import tpu as pltpu


def _body(a_ref, q_ref, i_ref, s_ref, o_ref, *, rep2, nb):
    q = q_ref[...]
    srep = jnp.repeat(s_ref[...], rep2, axis=0)
    b = ((q - 8).astype(jnp.float32) * srep).astype(jnp.bfloat16)
    ic = i_ref[...].astype(jnp.bfloat16)
    zero = jnp.zeros((), jnp.bfloat16)
    planes = [jnp.where(ic == p, b, zero) for p in range(4)]
    w = jnp.concatenate(planes, axis=0)
    o_ref[...] = jnp.dot(a_ref[...], w, preferred_element_type=jnp.float32)


def kernel(A, qvals, idx, s):
    M, K = A.shape
    K2, N = qvals.shape
    g4 = K2 // 2
    sg = s.shape[0]
    rep2 = K2 // sg
    nb = 256 if N % 256 == 0 else N

    Ac = jnp.transpose(A.reshape(M, g4, 4), (0, 2, 1)).reshape(M, K)
    A2 = jnp.repeat(Ac, 2, axis=1).astype(jnp.bfloat16)
    ic = jnp.transpose(idx, (0, 2, 1)).reshape(K2, N)

    body = functools.partial(_body, rep2=rep2, nb=nb)
    return pl.pallas_call(
        body,
        out_shape=jax.ShapeDtypeStruct((M, N), jnp.float32),
        grid=(N // nb,),
        in_specs=[
            pl.BlockSpec((M, 4 * K2), lambda n: (0, 0)),
            pl.BlockSpec((K2, nb), lambda n: (0, n)),
            pl.BlockSpec((K2, nb), lambda n: (0, n)),
            pl.BlockSpec((sg, nb), lambda n: (0, n)),
        ],
        out_specs=pl.BlockSpec((M, nb), lambda n: (0, n)),
        compiler_params=pltpu.CompilerParams(
            dimension_semantics=("parallel",),
            vmem_limit_bytes=50 * 1024 * 1024,
        ),
        name="sparse24_int4_matmul",
    )(A2, qvals, ic, s)
